# baseline (speedup 1.0000x reference)
import functools

import jax
import jax.numpy as jnp
from jax import lax
from jax.experimental import pallas as pl
from jax.experimental.pallas import tpu as pltpu
from jax.experimental.pallas import tpu_sc as plsc

N_VOCAB = 1_000_000
N_EMBED = 64
BATCH = 16384
N_SAMPLES = 3
N_ROWS = BATCH * N_SAMPLES

J_SC = 229_376
J_TC = N_VOCAB - J_SC
I_BLK = 8
J_BLK = 1024

SC_CHUNK = 8192
SC_UNROLL = 2

def _rotl(x, d):
    return lax.shift_left(x, jnp.int32(d)) | lax.shift_right_logical(
        x, jnp.int32(32 - d))


def _threefry_rounds(x0, x1, rots):
    for r in rots:
        x0 = x0 + x1
        x1 = _rotl(x1, r)
        x1 = x0 ^ x1
    return x0, x1


def _hash_bits(x0, x1):
    ks2 = jnp.int32(0x1BD11BDA ^ 42)
    r0 = (13, 15, 26, 6)
    r1 = (17, 29, 16, 24)
    x0, x1 = _threefry_rounds(x0, x1, r0)
    x0, x1 = x0 + jnp.int32(42), x1 + (ks2 + jnp.int32(1))
    x0, x1 = _threefry_rounds(x0, x1, r1)
    x0, x1 = x0 + ks2, x1 + jnp.int32(2)
    x0, x1 = _threefry_rounds(x0, x1, r0)
    x0, x1 = x0, x1 + jnp.int32(42 + 3)
    x0, x1 = _threefry_rounds(x0, x1, r1)
    x0, x1 = x0 + jnp.int32(42), x1 + (ks2 + jnp.int32(4))
    x0, x1 = _threefry_rounds(x0, x1, r0)
    x0, x1 = x0 + ks2, x1 + jnp.int32(5)
    return x0 ^ x1


def _u32(x):
    return lax.bitcast_convert_type(x, jnp.uint32)


def _sample_body(ip_ref, idx_ref, val_ref, *, n_vocab,
                 j_blk, j_trip, i_blk):
    pid = pl.program_id(0)
    row = lax.broadcasted_iota(jnp.int32, (i_blk, 1), 0)
    iglob = pid * i_blk + row
    base_lo = iglob * jnp.int32(n_vocab)
    base_hi = lax.shift_right_logical(iglob * jnp.int32(n_vocab // 64),
                                      jnp.int32(26))
    thresh = _u32(~base_lo)
    base_hi1 = base_hi + jnp.int32(1)

    lane = lax.broadcasted_iota(jnp.int32, (i_blk, j_blk), 1)
    e_base = (base_lo + jnp.int32(42)) + lane

    def body(jc, carry):
        run_val, run_idx = carry
        jb = jc * jnp.int32(j_blk)
        j = lane + jb
        x1 = e_base + jb
        carry_m = _u32(j) > thresh
        x0 = jnp.where(carry_m, base_hi1, base_hi)
        bits = _hash_bits(x0, x1)
        fm = lax.shift_right_logical(bits, jnp.int32(9)) | jnp.int32(
            0x3F800000)
        f = lax.bitcast_convert_type(fm, jnp.float32) - jnp.float32(1.0)
        val = jnp.log(f) * ip_ref[0, pl.ds(jc * j_blk, j_blk)][None, :]
        upd = val > run_val
        run_val = jnp.where(upd, val, run_val)
        run_idx = jnp.where(upd, j, run_idx)
        return run_val, run_idx

    init = (jnp.full((i_blk, j_blk), -jnp.inf, jnp.float32),
            jnp.zeros((i_blk, j_blk), jnp.int32))
    run_val, run_idx = lax.fori_loop(0, j_trip, body, init, unroll=2)
    m = jnp.max(run_val, axis=1, keepdims=True)
    cand = jnp.where(run_val == m, run_idx, jnp.int32(2147483647))
    idx_ref[0, :, :] = jnp.min(cand, axis=1, keepdims=True)
    val_ref[0, :, :] = m


def _make_tc_sampler(n_rows, n_classes, i_blk, j_blk):
    j_trip = -(-n_classes // j_blk)
    j_pad = j_trip * j_blk
    grid = n_rows // i_blk
    body = functools.partial(_sample_body, n_vocab=N_VOCAB, j_blk=j_blk,
                             j_trip=j_trip, i_blk=i_blk)
    return pl.pallas_call(
        body,
        grid=(grid,),
        in_specs=[pl.BlockSpec((1, j_pad), lambda i: (0, 0))],
        out_specs=[pl.BlockSpec((1, i_blk, 1), lambda i: (i, 0, 0)),
                   pl.BlockSpec((1, i_blk, 1), lambda i: (i, 0, 0))],
        out_shape=[jax.ShapeDtypeStruct((grid, i_blk, 1), jnp.int32),
                   jax.ShapeDtypeStruct((grid, i_blk, 1), jnp.float32)],
        compiler_params=pltpu.CompilerParams(
            dimension_semantics=("arbitrary",)),
    ), j_pad


def _ln_poly(f):
    z = lax.bitcast_convert_type(f, jnp.int32)
    e2 = lax.shift_right_logical(z, jnp.int32(23)) - jnp.int32(127)
    man = z & jnp.int32(0x7FFFFF)
    adj = man >= jnp.int32(0x3504F3)
    e2 = jnp.where(adj, e2 + jnp.int32(1), e2)
    g = lax.bitcast_convert_type(man | jnp.int32(0x3F800000), jnp.float32)
    g = jnp.where(adj, g * jnp.float32(0.5), g)
    r = g - jnp.float32(1.0)
    s = r / (g + jnp.float32(1.0))
    s2 = s * s
    p = jnp.float32(2.0 / 9.0)
    p = p * s2 + jnp.float32(2.0 / 7.0)
    p = p * s2 + jnp.float32(2.0 / 5.0)
    p = p * s2 + jnp.float32(2.0 / 3.0)
    p = p * s2 + jnp.float32(2.0)
    return e2.astype(jnp.float32) * jnp.float32(0.6931471805599453) + s * p


def _make_sc_sampler():
    info = plsc.get_sparse_core_info()
    nw = info.num_cores * info.num_subcores
    rows_w = N_ROWS // nw
    n_super = J_SC // SC_CHUNK
    j_inner = SC_CHUNK // (16 * SC_UNROLL)

    mesh = plsc.VectorSubcoreMesh(core_axis_name="c", subcore_axis_name="s")

    @functools.partial(
        pl.kernel,
        mesh=mesh,
        compiler_params=pltpu.CompilerParams(use_tc_tiling_on_sc=False,
                                             needs_layout_passes=False),
        out_type=[
            jax.ShapeDtypeStruct((N_ROWS,), jnp.float32),
            jax.ShapeDtypeStruct((N_ROWS,), jnp.int32),
        ],
        scratch_types=[
            pltpu.VMEM((SC_CHUNK,), jnp.float32),
            pltpu.VMEM((rows_w,), jnp.float32),
            pltpu.VMEM((rows_w,), jnp.int32),
            pltpu.VMEM((rows_w,), jnp.int32),
            pltpu.VMEM((rows_w,), jnp.int32),
            pltpu.VMEM((rows_w,), jnp.int32),
            pltpu.SemaphoreType.DMA,
        ],
    )
    def sampler(ip_hbm, blo_hbm, th_hbm, hi_hbm, val_out, idx_out,
                ipbuf, runv, runi, blov, thv, hiv, sem):
        wid = lax.axis_index("s") * info.num_cores + lax.axis_index("c")
        rowbase = wid * rows_w
        lane = lax.iota(jnp.int32, 16)
        lane0 = lane == jnp.int32(0)

        pltpu.sync_copy(blo_hbm.at[pl.ds(rowbase, rows_w)], blov)
        pltpu.sync_copy(th_hbm.at[pl.ds(rowbase, rows_w)], thv)
        pltpu.sync_copy(hi_hbm.at[pl.ds(rowbase, rows_w)], hiv)

        def init_g(g, _):
            runv[pl.ds(g * 16, 16)] = jnp.full((16,), -jnp.inf, jnp.float32)
            runi[pl.ds(g * 16, 16)] = jnp.zeros((16,), jnp.int32)
            return 0

        lax.fori_loop(0, rows_w // 16, init_g, 0)

        def super_body(sc, _):
            pltpu.sync_copy(ip_hbm.at[pl.ds(sc * SC_CHUNK, SC_CHUNK)], ipbuf)
            jsc0 = jnp.int32(J_TC) + sc * jnp.int32(SC_CHUNK)

            def row_body(r, _):
                rsp = jnp.full((16,), r, jnp.int32)
                blo42 = plsc.load_gather(blov, [rsp])
                thresh = _u32(plsc.load_gather(thv, [rsp]))
                base_hi = plsc.load_gather(hiv, [rsp])
                base_hi1 = base_hi + jnp.int32(1)
                rv = plsc.load_gather(runv, [rsp])
                ri = plsc.load_gather(runi, [rsp])

                def j_body(t, carry):
                    rv, ri = carry
                    for u in range(SC_UNROLL):
                        jloc = t * jnp.int32(16 * SC_UNROLL) + jnp.int32(
                            u * 16)
                        jv = (jsc0 + jloc) + lane
                        x1 = blo42 + jv
                        carry_m = _u32(jv) > thresh
                        x0 = jnp.where(carry_m, base_hi1, base_hi)
                        bits = _hash_bits(x0, x1)
                        fm = lax.shift_right_logical(
                            bits, jnp.int32(9)) | jnp.int32(0x3F800000)
                        f = lax.bitcast_convert_type(
                            fm, jnp.float32) - jnp.float32(1.0)
                        ip = ipbuf[pl.ds(jloc, 16)]
                        val = _ln_poly(f) * ip
                        upd = val > rv
                        rv = jnp.where(upd, val, rv)
                        ri = jnp.where(upd, jv, ri)
                    return rv, ri

                rv, ri = lax.fori_loop(0, j_inner, j_body, (rv, ri))
                m = jnp.max(rv)
                msp = jnp.full((16,), m, jnp.float32)
                cand = jnp.where(rv == msp, ri, jnp.int32(2147483647))
                mi = jnp.min(cand)
                plsc.store_scatter(runv, [rsp], msp, mask=lane0)
                plsc.store_scatter(runi, [rsp],
                                   jnp.full((16,), mi, jnp.int32),
                                   mask=lane0)
                return 0

            lax.fori_loop(0, rows_w, row_body, 0)
            return 0

        lax.fori_loop(0, n_super, super_body, 0)
        pltpu.sync_copy(runv, val_out.at[pl.ds(rowbase, rows_w)])
        pltpu.sync_copy(runi, idx_out.at[pl.ds(rowbase, rows_w)])

    return sampler


def _sc_gather_io(in_embed, out_embed, input_words, output_words):
    info = plsc.get_sparse_core_info()
    nw = info.num_cores * info.num_subcores
    b_io = BATCH // nw

    mesh = plsc.VectorSubcoreMesh(core_axis_name="c", subcore_axis_name="s")

    @functools.partial(
        pl.kernel,
        mesh=mesh,
        compiler_params=pltpu.CompilerParams(use_tc_tiling_on_sc=False),
        out_type=[
            jax.ShapeDtypeStruct((BATCH, N_EMBED), jnp.float32),
            jax.ShapeDtypeStruct((BATCH, N_EMBED), jnp.float32),
        ],
        scratch_types=[
            pltpu.VMEM((b_io,), jnp.int32),
            pltpu.VMEM((b_io, N_EMBED), jnp.float32),
            pltpu.SemaphoreType.DMA,
        ],
    )
    def gather(in_hbm, out_hbm, iw_hbm, ow_hbm, iv_out, ov_out, idx_v,
               rows_v, sem):
        wid = lax.axis_index("s") * info.num_cores + lax.axis_index("c")
        base = wid * b_io
        pltpu.sync_copy(iw_hbm.at[pl.ds(base, b_io)], idx_v)
        pltpu.async_copy(in_hbm.at[idx_v], rows_v, sem).wait()
        pltpu.sync_copy(rows_v, iv_out.at[pl.ds(base, b_io)])
        pltpu.sync_copy(ow_hbm.at[pl.ds(base, b_io)], idx_v)
        pltpu.async_copy(out_hbm.at[idx_v], rows_v, sem).wait()
        pltpu.sync_copy(rows_v, ov_out.at[pl.ds(base, b_io)])

    return gather(in_embed, out_embed, input_words, output_words)


def _sc_merge_gather_noise(out_embed, tc_val, tc_idx, sc_val, sc_idx):
    info = plsc.get_sparse_core_info()
    nw = info.num_cores * info.num_subcores
    b_nz = N_ROWS // nw
    n_chunk = b_nz // 512

    mesh = plsc.VectorSubcoreMesh(core_axis_name="c", subcore_axis_name="s")

    @functools.partial(
        pl.kernel,
        mesh=mesh,
        compiler_params=pltpu.CompilerParams(use_tc_tiling_on_sc=False),
        out_type=jax.ShapeDtypeStruct((N_ROWS, N_EMBED), jnp.float32),
        scratch_types=[
            pltpu.VMEM((b_nz,), jnp.float32),
            pltpu.VMEM((b_nz,), jnp.int32),
            pltpu.VMEM((b_nz,), jnp.float32),
            pltpu.VMEM((b_nz,), jnp.int32),
            pltpu.VMEM((512, N_EMBED), jnp.float32),
            pltpu.SemaphoreType.DMA,
        ],
    )
    def gather(out_hbm, tcv_hbm, tci_hbm, scv_hbm, sci_hbm, nv_out,
               tcv, tci, scv, sci, rows_v, sem):
        wid = lax.axis_index("s") * info.num_cores + lax.axis_index("c")
        base = wid * b_nz
        pltpu.sync_copy(tcv_hbm.at[pl.ds(base, b_nz)], tcv)
        pltpu.sync_copy(tci_hbm.at[pl.ds(base, b_nz)], tci)
        pltpu.sync_copy(scv_hbm.at[pl.ds(base, b_nz)], scv)
        pltpu.sync_copy(sci_hbm.at[pl.ds(base, b_nz)], sci)

        def merge_g(g, _):
            sl = pl.ds(g * 16, 16)
            pick_tc = tcv[sl] >= scv[sl]
            tci[sl] = jnp.where(pick_tc, tci[sl], sci[sl])
            return 0

        lax.fori_loop(0, b_nz // 16, merge_g, 0)
        for k in range(n_chunk):
            pltpu.async_copy(
                out_hbm.at[tci.at[pl.ds(k * 512, 512)]], rows_v, sem).wait()
            pltpu.sync_copy(rows_v,
                            nv_out.at[pl.ds(base + k * 512, 512)])

    return gather(out_embed, tc_val, tc_idx, sc_val, sc_idx)


def kernel(input_words, output_words, in_embed, out_embed, noise_dist):
    tc_sampler, j_pad = _make_tc_sampler(N_ROWS, J_TC, I_BLK, J_BLK)
    sc_sampler = _make_sc_sampler()
    ip = 1.0 / noise_dist
    ip_tc = jnp.concatenate(
        [ip[:J_TC], jnp.full((j_pad - J_TC,), 1e30, jnp.float32)])[None, :]
    ip_sc = ip[J_TC:]
    rows = jnp.arange(N_ROWS, dtype=jnp.int32)
    base_lo = rows * jnp.int32(N_VOCAB)
    blo42 = base_lo + jnp.int32(42)
    thresh = ~base_lo
    base_hi = jnp.right_shift(rows * jnp.int32(N_VOCAB // 64), 26)
    tc_idx, tc_val = tc_sampler(ip_tc)
    sc_val, sc_idx = sc_sampler(ip_sc, blo42, thresh, base_hi)
    iv, ov = _sc_gather_io(in_embed, out_embed,
                           input_words.astype(jnp.int32),
                           output_words.astype(jnp.int32))
    nv = _sc_merge_gather_noise(out_embed, tc_val.reshape(N_ROWS),
                                tc_idx.reshape(N_ROWS), sc_val, sc_idx)
    return iv, ov, nv.reshape(BATCH, N_SAMPLES, N_EMBED)
